# Initial kernel scaffold; baseline (speedup 1.0000x reference)
import jax
import jax.numpy as jnp
from jax.experimental import pallas as pl


def kernel(p, x, o, W, gamma, beta):
    raise NotImplementedError("write your pallas kernel here")


# speedup vs baseline: 3.8923x; 3.8923x over previous
import jax
import jax.numpy as jnp
from jax.experimental import pallas as pl
from jax.experimental.pallas import tpu as pltpu

B = 4
NPER = 25000
STRIDE = 16
M = NPER // STRIDE
NS = 16
CIN = 32
COUT = 64
CFEAT = 3 + CIN
LANE = 128
NPAD = ((NPER + LANE - 1) // LANE) * LANE
RQ = 8
MPAD = ((M + RQ - 1) // RQ) * RQ
NT = MPAD // RQ
ZCHUNK = NPAD // 8


def _fps_kernel(px_ref, py_ref, pz_ref, ox_ref, oy_ref, oz_ref, dists_ref):
    zeros_m = jnp.zeros((B, MPAD, 1), jnp.float32)
    ox_ref[...] = zeros_m
    oy_ref[...] = zeros_m
    oz_ref[...] = zeros_m
    px = px_ref[...]
    py = py_ref[...]
    pz = pz_ref[...]
    ox_ref[:, 0:1, :] = px[:, 0:1, None]
    oy_ref[:, 0:1, :] = py[:, 0:1, None]
    oz_ref[:, 0:1, :] = pz[:, 0:1, None]
    lane = jax.lax.broadcasted_iota(jnp.int32, (B, NPAD), 1)
    dists_ref[...] = jnp.where(lane < NPER, 1e10, -jnp.inf).astype(jnp.float32)

    def body(i, carry):
        qx, qy, qz = carry
        dx = px - qx
        dy = py - qy
        dz = pz - qz
        d = dx * dx + dy * dy + dz * dz
        dists = jnp.minimum(dists_ref[...], d)
        dists_ref[...] = dists
        maxv = jnp.max(dists, axis=1, keepdims=True)
        idx = jnp.min(jnp.where(dists == maxv, lane, NPAD), axis=1,
                      keepdims=True)
        hit = lane == idx
        nqx = jnp.sum(jnp.where(hit, px, 0.0), axis=1, keepdims=True)
        nqy = jnp.sum(jnp.where(hit, py, 0.0), axis=1, keepdims=True)
        nqz = jnp.sum(jnp.where(hit, pz, 0.0), axis=1, keepdims=True)
        ox_ref[:, pl.ds(i, 1), :] = nqx[:, :, None]
        oy_ref[:, pl.ds(i, 1), :] = nqy[:, :, None]
        oz_ref[:, pl.ds(i, 1), :] = nqz[:, :, None]
        return (nqx, nqy, nqz)

    q0 = (px[:, 0:1], py[:, 0:1], pz[:, 0:1])
    jax.lax.fori_loop(1, M, body, q0)


def _proj_kernel(feat_ref, wt_ref, z_ref):
    z_ref[0] = jnp.dot(feat_ref[0], wt_ref[...],
                       preferred_element_type=jnp.float32)


def _knn_kernel(px_ref, py_ref, pz_ref, q_ref, knn_ref, d2_ref):
    qx = q_ref[0, 0, :, 0:1]
    qy = q_ref[0, 0, :, 1:2]
    qz = q_ref[0, 0, :, 2:3]
    dx = qx - px_ref[0]
    dy = qy - py_ref[0]
    dz = qz - pz_ref[0]
    d2 = dx * dx + dy * dy + dz * dz
    lane = jax.lax.broadcasted_iota(jnp.int32, (RQ, NPAD), 1)
    d2_ref[...] = jnp.where(lane < NPER, d2, jnp.inf)
    siota = jax.lax.broadcasted_iota(jnp.int32, (RQ, NS), 1)

    def body(s, knn):
        d2c = d2_ref[...]
        minv = jnp.min(d2c, axis=1, keepdims=True)
        idx = jnp.min(jnp.where(d2c == minv, lane, NPAD), axis=1,
                      keepdims=True)
        d2_ref[...] = jnp.where(lane == idx, jnp.inf, d2c)
        return jnp.where(siota == s, idx, knn)

    knn = jax.lax.fori_loop(0, NS, body,
                            jnp.zeros((RQ, NS), jnp.int32))
    knn_ref[0, 0] = knn


def _gather_kernel(knn_ref, z_ref, zsum_ref, zsq_ref, zmax_ref, zmin_ref):
    sums, sqs, mxs, mns = [], [], [], []
    for q in range(RQ):
        r0 = z_ref[0, pl.ds(knn_ref[0, 0, q, 0], 1), :]
        asum, asq, amax, amin = r0, r0 * r0, r0, r0
        for s in range(1, NS):
            r = z_ref[0, pl.ds(knn_ref[0, 0, q, s], 1), :]
            asum = asum + r
            asq = asq + r * r
            amax = jnp.maximum(amax, r)
            amin = jnp.minimum(amin, r)
        sums.append(asum)
        sqs.append(asq)
        mxs.append(amax)
        mns.append(amin)
    zsum_ref[0, 0] = jnp.concatenate(sums, axis=0)
    zsq_ref[0, 0] = jnp.concatenate(sqs, axis=0)
    zmax_ref[0, 0] = jnp.concatenate(mxs, axis=0)
    zmin_ref[0, 0] = jnp.concatenate(mns, axis=0)


def _bn_kernel(zs_ref, zq_ref, zx_ref, zn_ref, npx_ref, npy_ref, npz_ref,
               wx_ref, wy_ref, wz_ref, g_ref, b_ref, out_ref):
    c = (npx_ref[...] * wx_ref[...] + npy_ref[...] * wy_ref[...]
         + npz_ref[...] * wz_ref[...])
    zs = zs_ref[...]
    hsum = zs - NS * c
    hsq = zq_ref[...] - 2.0 * c * zs + NS * (c * c)
    hmax = zx_ref[...] - c
    hmin = zn_ref[...] - c
    row = jax.lax.broadcasted_iota(jnp.int32, (B * MPAD, 1), 0)
    valid = (row % MPAD) < M
    cnt = float(B * M * NS)
    mean = jnp.sum(jnp.where(valid, hsum, 0.0), axis=0, keepdims=True) / cnt
    msq = jnp.sum(jnp.where(valid, hsq, 0.0), axis=0, keepdims=True) / cnt
    var = msq - mean * mean
    rstd = jax.lax.rsqrt(var + 1e-5)
    g = g_ref[...]
    hsel = jnp.where(g >= 0.0, hmax, hmin)
    out_ref[...] = jnp.maximum((hsel - mean) * rstd * g + b_ref[...], 0.0)


def kernel(p, x, o, W, gamma, beta):
    p4 = p.reshape(B, NPER, 3)
    x4 = x.reshape(B, NPER, CIN)
    p_pad = jnp.pad(p4, ((0, 0), (0, NPAD - NPER), (0, 0)))
    x_pad = jnp.pad(x4, ((0, 0), (0, NPAD - NPER), (0, 0)))
    px = p_pad[..., 0]
    py = p_pad[..., 1]
    pz = p_pad[..., 2]

    f32 = jnp.float32
    ox, oy, oz = pl.pallas_call(
        _fps_kernel,
        out_shape=[jax.ShapeDtypeStruct((B, MPAD, 1), f32)] * 3,
        scratch_shapes=[pltpu.VMEM((B, NPAD), f32)],
    )(px, py, pz)
    npc = jnp.concatenate([ox, oy, oz], axis=-1)
    n_p = npc[:, :M, :].reshape(B * M, 3)

    feat = jnp.concatenate([p_pad, x_pad], axis=-1)
    z = pl.pallas_call(
        _proj_kernel,
        grid=(B, NPAD // ZCHUNK),
        in_specs=[
            pl.BlockSpec((1, ZCHUNK, CFEAT), lambda b, c: (b, c, 0)),
            pl.BlockSpec((CFEAT, COUT), lambda b, c: (0, 0)),
        ],
        out_specs=pl.BlockSpec((1, ZCHUNK, COUT), lambda b, c: (b, c, 0)),
        out_shape=jax.ShapeDtypeStruct((B, NPAD, COUT), f32),
    )(feat, W.T)

    qtiles = npc.reshape(B, NT, RQ, 3)
    knn = pl.pallas_call(
        _knn_kernel,
        grid=(B, NT),
        in_specs=[
            pl.BlockSpec((1, 1, NPAD), lambda b, t: (b, 0, 0)),
            pl.BlockSpec((1, 1, NPAD), lambda b, t: (b, 0, 0)),
            pl.BlockSpec((1, 1, NPAD), lambda b, t: (b, 0, 0)),
            pl.BlockSpec((1, 1, RQ, 3), lambda b, t: (b, t, 0, 0)),
        ],
        out_specs=pl.BlockSpec((1, 1, RQ, NS), lambda b, t: (b, t, 0, 0)),
        out_shape=jax.ShapeDtypeStruct((B, NT, RQ, NS), jnp.int32),
        scratch_shapes=[pltpu.VMEM((RQ, NPAD), f32)],
    )(px[:, None, :], py[:, None, :], pz[:, None, :], qtiles)

    group_specs = pl.BlockSpec((1, 1, RQ, COUT), lambda b, t: (b, t, 0, 0))
    group_shape = jax.ShapeDtypeStruct((B, NT, RQ, COUT), f32)
    zsum, zsq, zmax, zmin = pl.pallas_call(
        _gather_kernel,
        grid=(B, NT),
        in_specs=[
            pl.BlockSpec((1, 1, RQ, NS), lambda b, t: (b, t, 0, 0),
                         memory_space=pltpu.SMEM),
            pl.BlockSpec((1, NPAD, COUT), lambda b, t: (b, 0, 0)),
        ],
        out_specs=[group_specs] * 4,
        out_shape=[group_shape] * 4,
    )(knn, z)

    flat = lambda a: a.reshape(B * MPAD, COUT)
    out_full = pl.pallas_call(
        _bn_kernel,
        out_shape=jax.ShapeDtypeStruct((B * MPAD, COUT), f32),
    )(flat(zsum), flat(zsq), flat(zmax), flat(zmin),
      ox.reshape(B * MPAD, 1), oy.reshape(B * MPAD, 1),
      oz.reshape(B * MPAD, 1),
      W[:, 0].reshape(1, COUT), W[:, 1].reshape(1, COUT),
      W[:, 2].reshape(1, COUT),
      gamma.reshape(1, COUT), beta.reshape(1, COUT))
    out = out_full.reshape(B, MPAD, COUT)[:, :M, :].reshape(B * M, COUT)

    n_o = jnp.arange(1, B + 1, dtype=jnp.int32) * M
    return (n_p, out, n_o)


# speedup vs baseline: 6.4832x; 1.6656x over previous
import jax
import jax.numpy as jnp
from jax.experimental import pallas as pl
from jax.experimental.pallas import tpu as pltpu

B = 4
NPER = 25000
STRIDE = 16
M = NPER // STRIDE
NS = 16
CIN = 32
COUT = 64
CFEAT = 3 + CIN
LANE = 128
NPAD = ((NPER + LANE - 1) // LANE) * LANE
RQ = 32
MPAD = 1568
NT = MPAD // RQ
GQ = 8
GT = MPAD // GQ
ZCHUNK = NPAD // 8
NHALF = NPAD // 2


def _fps_kernel(px_ref, py_ref, pz_ref, ox_ref, oy_ref, oz_ref, dists_ref):
    B2 = 2 * B
    zeros_m = jnp.zeros((B2, MPAD, 1), jnp.float32)
    ox_ref[...] = zeros_m
    oy_ref[...] = zeros_m
    oz_ref[...] = zeros_m
    px = px_ref[...]
    py = py_ref[...]
    pz = pz_ref[...]
    lane = jax.lax.broadcasted_iota(jnp.int32, (B2, NHALF), 1)
    sub = jax.lax.broadcasted_iota(jnp.int32, (B2, NHALF), 0)
    gidx = lane + (sub % 2) * NHALF
    even = (jax.lax.broadcasted_iota(jnp.int32, (B2, 1), 0) % 2) == 0

    def pair(x, op):
        y = op(x, jnp.roll(x, -1, axis=0))
        return jnp.where(even, y, jnp.roll(y, 1, axis=0))

    dists_ref[...] = jnp.where(gidx < NPER, 1e10, -jnp.inf).astype(jnp.float32)

    def extract(hit):
        nqx = pair(jnp.sum(jnp.where(hit, px, 0.0), axis=1, keepdims=True),
                   jnp.add)
        nqy = pair(jnp.sum(jnp.where(hit, py, 0.0), axis=1, keepdims=True),
                   jnp.add)
        nqz = pair(jnp.sum(jnp.where(hit, pz, 0.0), axis=1, keepdims=True),
                   jnp.add)
        return nqx, nqy, nqz

    q0 = extract(gidx == 0)
    ox_ref[:, 0:1, :] = q0[0][:, :, None]
    oy_ref[:, 0:1, :] = q0[1][:, :, None]
    oz_ref[:, 0:1, :] = q0[2][:, :, None]

    def body(i, carry):
        qx, qy, qz = carry
        dx = px - qx
        dy = py - qy
        dz = pz - qz
        d = dx * dx + dy * dy + dz * dz
        dists = jnp.minimum(dists_ref[...], d)
        dists_ref[...] = dists
        maxv = pair(jnp.max(dists, axis=1, keepdims=True), jnp.maximum)
        idx = pair(jnp.min(jnp.where(dists == maxv, gidx, NPAD), axis=1,
                           keepdims=True), jnp.minimum)
        nq = extract(gidx == idx)
        ox_ref[:, pl.ds(i, 1), :] = nq[0][:, :, None]
        oy_ref[:, pl.ds(i, 1), :] = nq[1][:, :, None]
        oz_ref[:, pl.ds(i, 1), :] = nq[2][:, :, None]
        return nq

    jax.lax.fori_loop(1, M, body, q0)


def _proj_kernel(feat_ref, wt_ref, z_ref):
    z_ref[0] = jnp.dot(feat_ref[0], wt_ref[...],
                       preferred_element_type=jnp.float32)


def _knn_kernel(px_ref, py_ref, pz_ref, q_ref, knn_ref, d2_ref):
    qx = q_ref[0, 0, :, 0:1]
    qy = q_ref[0, 0, :, 1:2]
    qz = q_ref[0, 0, :, 2:3]
    dx = qx - px_ref[0]
    dy = qy - py_ref[0]
    dz = qz - pz_ref[0]
    d2 = dx * dx + dy * dy + dz * dz
    lane = jax.lax.broadcasted_iota(jnp.int32, (RQ, NPAD), 1)
    d2_ref[...] = jnp.where(lane < NPER, d2, jnp.inf)
    siota = jax.lax.broadcasted_iota(jnp.int32, (RQ, NS), 1)

    def body(s, knn):
        d2c = d2_ref[...]
        minv = jnp.min(d2c, axis=1, keepdims=True)
        idx = jnp.min(jnp.where(d2c == minv, lane, NPAD), axis=1,
                      keepdims=True)
        d2_ref[...] = jnp.where(lane == idx, jnp.inf, d2c)
        return jnp.where(siota == s, idx, knn)

    knn = jax.lax.fori_loop(0, NS, body,
                            jnp.zeros((RQ, NS), jnp.int32))
    knn_ref[0, 0] = knn


def _gather_kernel(knn_ref, z_ref, zsum_ref, zsq_ref, zmax_ref, zmin_ref):
    sums, sqs, mxs, mns = [], [], [], []
    for q in range(GQ):
        r0 = z_ref[0, pl.ds(knn_ref[0, 0, q, 0], 1), :]
        asum, asq, amax, amin = r0, r0 * r0, r0, r0
        for s in range(1, NS):
            r = z_ref[0, pl.ds(knn_ref[0, 0, q, s], 1), :]
            asum = asum + r
            asq = asq + r * r
            amax = jnp.maximum(amax, r)
            amin = jnp.minimum(amin, r)
        sums.append(asum)
        sqs.append(asq)
        mxs.append(amax)
        mns.append(amin)
    zsum_ref[0, 0] = jnp.concatenate(sums, axis=0)
    zsq_ref[0, 0] = jnp.concatenate(sqs, axis=0)
    zmax_ref[0, 0] = jnp.concatenate(mxs, axis=0)
    zmin_ref[0, 0] = jnp.concatenate(mns, axis=0)


def _bn_kernel(zs_ref, zq_ref, zx_ref, zn_ref, npx_ref, npy_ref, npz_ref,
               wx_ref, wy_ref, wz_ref, g_ref, b_ref, out_ref):
    c = (npx_ref[...] * wx_ref[...] + npy_ref[...] * wy_ref[...]
         + npz_ref[...] * wz_ref[...])
    zs = zs_ref[...]
    hsum = zs - NS * c
    hsq = zq_ref[...] - 2.0 * c * zs + NS * (c * c)
    hmax = zx_ref[...] - c
    hmin = zn_ref[...] - c
    row = jax.lax.broadcasted_iota(jnp.int32, (B * MPAD, 1), 0)
    valid = (row % MPAD) < M
    cnt = float(B * M * NS)
    mean = jnp.sum(jnp.where(valid, hsum, 0.0), axis=0, keepdims=True) / cnt
    msq = jnp.sum(jnp.where(valid, hsq, 0.0), axis=0, keepdims=True) / cnt
    var = msq - mean * mean
    rstd = jax.lax.rsqrt(var + 1e-5)
    g = g_ref[...]
    hsel = jnp.where(g >= 0.0, hmax, hmin)
    out_ref[...] = jnp.maximum((hsel - mean) * rstd * g + b_ref[...], 0.0)


def kernel(p, x, o, W, gamma, beta):
    p4 = p.reshape(B, NPER, 3)
    x4 = x.reshape(B, NPER, CIN)
    p_pad = jnp.pad(p4, ((0, 0), (0, NPAD - NPER), (0, 0)))
    x_pad = jnp.pad(x4, ((0, 0), (0, NPAD - NPER), (0, 0)))
    px = p_pad[..., 0]
    py = p_pad[..., 1]
    pz = p_pad[..., 2]

    f32 = jnp.float32
    fold = lambda a: a.reshape(2 * B, NHALF)
    ox8, oy8, oz8 = pl.pallas_call(
        _fps_kernel,
        out_shape=[jax.ShapeDtypeStruct((2 * B, MPAD, 1), f32)] * 3,
        scratch_shapes=[pltpu.VMEM((2 * B, NHALF), f32)],
    )(fold(px), fold(py), fold(pz))
    ox, oy, oz = ox8[0::2], oy8[0::2], oz8[0::2]
    npc = jnp.concatenate([ox, oy, oz], axis=-1)
    n_p = npc[:, :M, :].reshape(B * M, 3)

    feat = jnp.concatenate([p_pad, x_pad], axis=-1)
    z = pl.pallas_call(
        _proj_kernel,
        grid=(B, NPAD // ZCHUNK),
        in_specs=[
            pl.BlockSpec((1, ZCHUNK, CFEAT), lambda b, c: (b, c, 0)),
            pl.BlockSpec((CFEAT, COUT), lambda b, c: (0, 0)),
        ],
        out_specs=pl.BlockSpec((1, ZCHUNK, COUT), lambda b, c: (b, c, 0)),
        out_shape=jax.ShapeDtypeStruct((B, NPAD, COUT), f32),
    )(feat, W.T)

    qtiles = npc.reshape(B, NT, RQ, 3)
    knn = pl.pallas_call(
        _knn_kernel,
        grid=(B, NT),
        in_specs=[
            pl.BlockSpec((1, 1, NPAD), lambda b, t: (b, 0, 0)),
            pl.BlockSpec((1, 1, NPAD), lambda b, t: (b, 0, 0)),
            pl.BlockSpec((1, 1, NPAD), lambda b, t: (b, 0, 0)),
            pl.BlockSpec((1, 1, RQ, 3), lambda b, t: (b, t, 0, 0)),
        ],
        out_specs=pl.BlockSpec((1, 1, RQ, NS), lambda b, t: (b, t, 0, 0)),
        out_shape=jax.ShapeDtypeStruct((B, NT, RQ, NS), jnp.int32),
        scratch_shapes=[pltpu.VMEM((RQ, NPAD), f32)],
    )(px[:, None, :], py[:, None, :], pz[:, None, :], qtiles)

    group_specs = pl.BlockSpec((1, 1, GQ, COUT), lambda b, t: (b, t, 0, 0))
    group_shape = jax.ShapeDtypeStruct((B, GT, GQ, COUT), f32)
    zsum, zsq, zmax, zmin = pl.pallas_call(
        _gather_kernel,
        grid=(B, GT),
        in_specs=[
            pl.BlockSpec((1, 1, GQ, NS), lambda b, t: (b, t, 0, 0),
                         memory_space=pltpu.SMEM),
            pl.BlockSpec((1, NPAD, COUT), lambda b, t: (b, 0, 0)),
        ],
        out_specs=[group_specs] * 4,
        out_shape=[group_shape] * 4,
    )(knn.reshape(B, GT, GQ, NS), z)

    flat = lambda a: a.reshape(B * MPAD, COUT)
    out_full = pl.pallas_call(
        _bn_kernel,
        out_shape=jax.ShapeDtypeStruct((B * MPAD, COUT), f32),
    )(flat(zsum), flat(zsq), flat(zmax), flat(zmin),
      ox.reshape(B * MPAD, 1), oy.reshape(B * MPAD, 1),
      oz.reshape(B * MPAD, 1),
      W[:, 0].reshape(1, COUT), W[:, 1].reshape(1, COUT),
      W[:, 2].reshape(1, COUT),
      gamma.reshape(1, COUT), beta.reshape(1, COUT))
    out = out_full.reshape(B, MPAD, COUT)[:, :M, :].reshape(B * M, COUT)

    n_o = jnp.arange(1, B + 1, dtype=jnp.int32) * M
    return (n_p, out, n_o)


# speedup vs baseline: 9.6270x; 1.4849x over previous
import jax
import jax.numpy as jnp
from jax.experimental import pallas as pl
from jax.experimental.pallas import tpu as pltpu

B = 4
NPER = 25000
STRIDE = 16
M = NPER // STRIDE
NS = 16
CIN = 32
COUT = 64
CFEAT = 3 + CIN
LANE = 128
NPAD = ((NPER + LANE - 1) // LANE) * LANE
RQ = 32
MPAD = 1568
NT = MPAD // RQ
GQ = 8
GT = MPAD // GQ
ZCHUNK = NPAD // 8
NHALF = NPAD // 2


def _fps_kernel(px_ref, py_ref, pz_ref, ox_ref, oy_ref, oz_ref, dists_ref):
    B2 = 2 * B
    zeros_m = jnp.zeros((B2, MPAD, 1), jnp.float32)
    ox_ref[...] = zeros_m
    oy_ref[...] = zeros_m
    oz_ref[...] = zeros_m
    px = px_ref[...]
    py = py_ref[...]
    pz = pz_ref[...]
    lane = jax.lax.broadcasted_iota(jnp.int32, (B2, NHALF), 1)
    sub = jax.lax.broadcasted_iota(jnp.int32, (B2, NHALF), 0)
    gidx = lane + (sub % 2) * NHALF
    even = (jax.lax.broadcasted_iota(jnp.int32, (B2, 1), 0) % 2) == 0

    def pair(x, op):
        y = op(x, jnp.roll(x, -1, axis=0))
        return jnp.where(even, y, jnp.roll(y, 1, axis=0))

    dists_ref[...] = jnp.where(gidx < NPER, 1e10, -jnp.inf).astype(jnp.float32)

    def extract(hit):
        nqx = pair(jnp.sum(jnp.where(hit, px, 0.0), axis=1, keepdims=True),
                   jnp.add)
        nqy = pair(jnp.sum(jnp.where(hit, py, 0.0), axis=1, keepdims=True),
                   jnp.add)
        nqz = pair(jnp.sum(jnp.where(hit, pz, 0.0), axis=1, keepdims=True),
                   jnp.add)
        return nqx, nqy, nqz

    q0 = extract(gidx == 0)
    ox_ref[:, 0:1, :] = q0[0][:, :, None]
    oy_ref[:, 0:1, :] = q0[1][:, :, None]
    oz_ref[:, 0:1, :] = q0[2][:, :, None]

    def body(i, carry):
        qx, qy, qz = carry
        dx = px - qx
        dy = py - qy
        dz = pz - qz
        d = dx * dx + dy * dy + dz * dz
        dists = jnp.minimum(dists_ref[...], d)
        dists_ref[...] = dists
        maxv = pair(jnp.max(dists, axis=1, keepdims=True), jnp.maximum)
        idx = pair(jnp.min(jnp.where(dists == maxv, gidx, NPAD), axis=1,
                           keepdims=True), jnp.minimum)
        nq = extract(gidx == idx)
        ox_ref[:, pl.ds(i, 1), :] = nq[0][:, :, None]
        oy_ref[:, pl.ds(i, 1), :] = nq[1][:, :, None]
        oz_ref[:, pl.ds(i, 1), :] = nq[2][:, :, None]
        return nq

    jax.lax.fori_loop(1, M, body, q0)


def _proj_kernel(feat_ref, wt_ref, z_ref):
    z_ref[0] = jnp.dot(feat_ref[0], wt_ref[...],
                       preferred_element_type=jnp.float32)


CCH = NPAD // LANE
K1 = 6


def _knn_kernel(px_ref, py_ref, pz_ref, q_ref, knn_ref, d2_ref):
    qx = q_ref[0, 0, :, 0:1][:, :, None]
    qy = q_ref[0, 0, :, 1:2][:, :, None]
    qz = q_ref[0, 0, :, 2:3][:, :, None]
    dx = qx - px_ref[...]
    dy = qy - py_ref[...]
    dz = qz - pz_ref[...]
    d2 = dx * dx + dy * dy + dz * dz
    sub = jax.lax.broadcasted_iota(jnp.int32, (RQ, CCH, LANE), 1)
    lan = jax.lax.broadcasted_iota(jnp.int32, (RQ, CCH, LANE), 2)
    gidx = sub * LANE + lan
    d2_ref[...] = jnp.where(gidx < NPER, d2, jnp.inf)

    cvals, crows = [], []
    for r in range(K1):
        d2c = d2_ref[...]
        minv = jnp.min(d2c, axis=1, keepdims=True)
        rsel = jnp.min(jnp.where(d2c == minv, sub, CCH), axis=1,
                       keepdims=True)
        if r + 1 < K1:
            d2_ref[...] = jnp.where(sub == rsel, jnp.inf, d2c)
        cvals.append(minv)
        crows.append(rsel)
    cval = jnp.concatenate(cvals, axis=1)
    crow = jnp.concatenate(crows, axis=1)
    cgid = crow * LANE + jax.lax.broadcasted_iota(jnp.int32, (RQ, K1, LANE), 2)
    cvalf = cval.reshape(RQ, K1 * LANE)
    cgidf = cgid.reshape(RQ, K1 * LANE)

    siota = jax.lax.broadcasted_iota(jnp.int32, (RQ, NS), 1)
    BIGI = jnp.int32(2**30)

    def body(s, carry):
        vals, knn = carry
        minv = jnp.min(vals, axis=1, keepdims=True)
        idx = jnp.min(jnp.where(vals == minv, cgidf, BIGI), axis=1,
                      keepdims=True)
        vals = jnp.where(cgidf == idx, jnp.inf, vals)
        return (vals, jnp.where(siota == s, idx, knn))

    _, knn = jax.lax.fori_loop(0, NS, body,
                               (cvalf, jnp.zeros((RQ, NS), jnp.int32)))
    knn_ref[0, 0] = knn


def _gather_kernel(knn_ref, z_ref, zsum_ref, zsq_ref, zmax_ref, zmin_ref):
    sums, sqs, mxs, mns = [], [], [], []
    for q in range(GQ):
        r0 = z_ref[0, pl.ds(knn_ref[0, 0, q, 0], 1), :]
        asum, asq, amax, amin = r0, r0 * r0, r0, r0
        for s in range(1, NS):
            r = z_ref[0, pl.ds(knn_ref[0, 0, q, s], 1), :]
            asum = asum + r
            asq = asq + r * r
            amax = jnp.maximum(amax, r)
            amin = jnp.minimum(amin, r)
        sums.append(asum)
        sqs.append(asq)
        mxs.append(amax)
        mns.append(amin)
    zsum_ref[0, 0] = jnp.concatenate(sums, axis=0)
    zsq_ref[0, 0] = jnp.concatenate(sqs, axis=0)
    zmax_ref[0, 0] = jnp.concatenate(mxs, axis=0)
    zmin_ref[0, 0] = jnp.concatenate(mns, axis=0)


def _bn_kernel(zs_ref, zq_ref, zx_ref, zn_ref, npx_ref, npy_ref, npz_ref,
               wx_ref, wy_ref, wz_ref, g_ref, b_ref, out_ref):
    c = (npx_ref[...] * wx_ref[...] + npy_ref[...] * wy_ref[...]
         + npz_ref[...] * wz_ref[...])
    zs = zs_ref[...]
    hsum = zs - NS * c
    hsq = zq_ref[...] - 2.0 * c * zs + NS * (c * c)
    hmax = zx_ref[...] - c
    hmin = zn_ref[...] - c
    row = jax.lax.broadcasted_iota(jnp.int32, (B * MPAD, 1), 0)
    valid = (row % MPAD) < M
    cnt = float(B * M * NS)
    mean = jnp.sum(jnp.where(valid, hsum, 0.0), axis=0, keepdims=True) / cnt
    msq = jnp.sum(jnp.where(valid, hsq, 0.0), axis=0, keepdims=True) / cnt
    var = msq - mean * mean
    rstd = jax.lax.rsqrt(var + 1e-5)
    g = g_ref[...]
    hsel = jnp.where(g >= 0.0, hmax, hmin)
    out_ref[...] = jnp.maximum((hsel - mean) * rstd * g + b_ref[...], 0.0)


def kernel(p, x, o, W, gamma, beta):
    p4 = p.reshape(B, NPER, 3)
    x4 = x.reshape(B, NPER, CIN)
    p_pad = jnp.pad(p4, ((0, 0), (0, NPAD - NPER), (0, 0)))
    x_pad = jnp.pad(x4, ((0, 0), (0, NPAD - NPER), (0, 0)))
    px = p_pad[..., 0]
    py = p_pad[..., 1]
    pz = p_pad[..., 2]

    f32 = jnp.float32
    fold = lambda a: a.reshape(2 * B, NHALF)
    ox8, oy8, oz8 = pl.pallas_call(
        _fps_kernel,
        out_shape=[jax.ShapeDtypeStruct((2 * B, MPAD, 1), f32)] * 3,
        scratch_shapes=[pltpu.VMEM((2 * B, NHALF), f32)],
    )(fold(px), fold(py), fold(pz))
    ox, oy, oz = ox8[0::2], oy8[0::2], oz8[0::2]
    npc = jnp.concatenate([ox, oy, oz], axis=-1)
    n_p = npc[:, :M, :].reshape(B * M, 3)

    feat = jnp.concatenate([p_pad, x_pad], axis=-1)
    z = pl.pallas_call(
        _proj_kernel,
        grid=(B, NPAD // ZCHUNK),
        in_specs=[
            pl.BlockSpec((1, ZCHUNK, CFEAT), lambda b, c: (b, c, 0)),
            pl.BlockSpec((CFEAT, COUT), lambda b, c: (0, 0)),
        ],
        out_specs=pl.BlockSpec((1, ZCHUNK, COUT), lambda b, c: (b, c, 0)),
        out_shape=jax.ShapeDtypeStruct((B, NPAD, COUT), f32),
    )(feat, W.T)

    qtiles = npc.reshape(B, NT, RQ, 3)
    knn = pl.pallas_call(
        _knn_kernel,
        grid=(B, NT),
        in_specs=[
            pl.BlockSpec((1, CCH, LANE), lambda b, t: (b, 0, 0)),
            pl.BlockSpec((1, CCH, LANE), lambda b, t: (b, 0, 0)),
            pl.BlockSpec((1, CCH, LANE), lambda b, t: (b, 0, 0)),
            pl.BlockSpec((1, 1, RQ, 3), lambda b, t: (b, t, 0, 0)),
        ],
        out_specs=pl.BlockSpec((1, 1, RQ, NS), lambda b, t: (b, t, 0, 0)),
        out_shape=jax.ShapeDtypeStruct((B, NT, RQ, NS), jnp.int32),
        scratch_shapes=[pltpu.VMEM((RQ, CCH, LANE), f32)],
    )(px.reshape(B, CCH, LANE), py.reshape(B, CCH, LANE),
      pz.reshape(B, CCH, LANE), qtiles)

    group_specs = pl.BlockSpec((1, 1, GQ, COUT), lambda b, t: (b, t, 0, 0))
    group_shape = jax.ShapeDtypeStruct((B, GT, GQ, COUT), f32)
    zsum, zsq, zmax, zmin = pl.pallas_call(
        _gather_kernel,
        grid=(B, GT),
        in_specs=[
            pl.BlockSpec((1, 1, GQ, NS), lambda b, t: (b, t, 0, 0),
                         memory_space=pltpu.SMEM),
            pl.BlockSpec((1, NPAD, COUT), lambda b, t: (b, 0, 0)),
        ],
        out_specs=[group_specs] * 4,
        out_shape=[group_shape] * 4,
    )(knn.reshape(B, GT, GQ, NS), z)

    flat = lambda a: a.reshape(B * MPAD, COUT)
    out_full = pl.pallas_call(
        _bn_kernel,
        out_shape=jax.ShapeDtypeStruct((B * MPAD, COUT), f32),
    )(flat(zsum), flat(zsq), flat(zmax), flat(zmin),
      ox.reshape(B * MPAD, 1), oy.reshape(B * MPAD, 1),
      oz.reshape(B * MPAD, 1),
      W[:, 0].reshape(1, COUT), W[:, 1].reshape(1, COUT),
      W[:, 2].reshape(1, COUT),
      gamma.reshape(1, COUT), beta.reshape(1, COUT))
    out = out_full.reshape(B, MPAD, COUT)[:, :M, :].reshape(B * M, COUT)

    n_o = jnp.arange(1, B + 1, dtype=jnp.int32) * M
    return (n_p, out, n_o)
